# baseline (speedup 1.0000x reference)
import functools

import jax
import jax.numpy as jnp
from jax import lax
from jax.experimental import pallas as pl
from jax.experimental.pallas import tpu as pltpu
from jax.experimental.pallas import tpu_sc as plsc

_NC = 2
_NS = 16
_NW = _NC * _NS
_NBUF = 2


def _gather_kernel(idx_hbm, table_hbm, out_hbm, idx_v, bufs, sem_g, sem_w):
    b_per_w, hist = idx_v.shape
    nr = b_per_w // _NBUF
    wid = lax.axis_index("s") * _NC + lax.axis_index("c")
    b0 = wid * b_per_w
    pltpu.sync_copy(idx_hbm.at[pl.ds(b0, b_per_w)], idx_v)

    def idx_slice(t):
        return idx_v.at[t]

    def out_slice(t):
        return out_hbm.at[b0 + t]

    def fire_g(slot, t):
        pltpu.async_copy(table_hbm.at[idx_slice(t)], bufs.at[slot], sem_g.at[slot])

    def wait_g(slot, t):
        pltpu.make_async_copy(
            table_hbm.at[idx_slice(t)], bufs.at[slot], sem_g.at[slot]
        ).wait()

    def fire_w(slot, t):
        pltpu.async_copy(bufs.at[slot], out_slice(t), sem_w.at[slot])

    def wait_w(slot, t):
        pltpu.make_async_copy(bufs.at[slot], out_slice(t), sem_w.at[slot]).wait()

    for b in range(_NBUF):
        fire_g(b, b)
    for b in range(_NBUF):
        wait_g(b, b)
        fire_w(b, b)
    for b in range(_NBUF):
        fire_g(_NBUF + b, _NBUF + b)

    @pl.loop(1, nr - 1, step=2)
    def _steady(r):
        for b in range(_NBUF):
            wait_g(_NBUF + b, r * _NBUF + b)
            fire_w(_NBUF + b, r * _NBUF + b)
        for b in range(_NBUF):
            wait_w(b, (r - 1) * _NBUF + b)
            fire_g(b, (r + 1) * _NBUF + b)
        for b in range(_NBUF):
            wait_g(b, (r + 1) * _NBUF + b)
            fire_w(b, (r + 1) * _NBUF + b)
        for b in range(_NBUF):
            wait_w(_NBUF + b, r * _NBUF + b)
            fire_g(_NBUF + b, (r + 2) * _NBUF + b)

    for b in range(_NBUF):
        wait_g(_NBUF + b, (nr - 1) * _NBUF + b)
        fire_w(_NBUF + b, (nr - 1) * _NBUF + b)
    for b in range(_NBUF):
        wait_w(b, (nr - 2) * _NBUF + b)
    for b in range(_NBUF):
        wait_w(_NBUF + b, (nr - 1) * _NBUF + b)


def kernel(input, table):
    batch, hist = input.shape
    _, embed_dim = table.shape
    assert batch % _NW == 0
    b_per_w = batch // _NW
    assert b_per_w % (2 * _NBUF) == 0

    run = functools.partial(
        pl.kernel,
        out_type=jax.ShapeDtypeStruct((batch, hist, embed_dim), table.dtype),
        mesh=plsc.VectorSubcoreMesh(core_axis_name="c", subcore_axis_name="s"),
        scratch_types=[
            pltpu.VMEM((b_per_w, hist), jnp.int32),
            pltpu.VMEM((2 * _NBUF, hist, embed_dim), jnp.float32),
            pltpu.SemaphoreType.DMA((2 * _NBUF,)),
            pltpu.SemaphoreType.DMA((2 * _NBUF,)),
        ],
        compiler_params=pltpu.CompilerParams(use_tc_tiling_on_sc=False),
    )(_gather_kernel)

    return run(input, table)
